# baseline (speedup 1.0000x reference)
import functools

import jax
import jax.numpy as jnp
from jax import lax
from jax.experimental import pallas as pl
from jax.experimental.pallas import tpu as pltpu
from jax.experimental.pallas import tpu_sc as plsc

NUM_CLASSES_ = 1000
N_ = 16384
NC_ = 2
NS_ = 16
L_ = 16
NW_ = NC_ * NS_
RPW_ = N_ // NW_
SUB_ = 16
FB_ = SUB_ * NUM_CLASSES_
NB_ = 4
ITERS_ = RPW_ // SUB_


def _sc_onehot(idx_hbm, zeros_hbm, out_hbm, idx_v, bufs, zsem, sems):
    wid = lax.axis_index("s") * NC_ + lax.axis_index("c")
    base = wid * RPW_
    pltpu.sync_copy(idx_hbm.at[pl.ds(base, RPW_)], idx_v)
    pltpu.make_async_copy(zeros_hbm, bufs, zsem).start()

    zeros16 = jnp.zeros((L_,), jnp.int32)
    ones16 = jnp.ones((L_,), jnp.int32)
    rows16 = lax.iota(jnp.int32, L_)

    def copies(b, t):
        return [
            pltpu.make_async_copy(
                bufs.at[b],
                out_hbm.at[pl.ds(base + t * SUB_, SUB_), :],
                sems.at[b],
            )
        ]

    pltpu.make_async_copy(zeros_hbm, bufs, zsem).wait()

    for t in range(ITERS_):
        b = t % NB_
        if t >= NB_:
            for cp in copies(b, t - NB_):
                cp.wait()
            for k in range(SUB_ // L_):
                col16 = idx_v[pl.ds((t - NB_) * SUB_ + k * L_, L_)]
                plsc.store_scatter(bufs.at[b], [rows16 + k * L_, col16], zeros16)
        for k in range(SUB_ // L_):
            col16 = idx_v[pl.ds(t * SUB_ + k * L_, L_)]
            plsc.store_scatter(bufs.at[b], [rows16 + k * L_, col16], ones16)
        for cp in copies(b, t):
            cp.start()

    for t in range(ITERS_ - NB_, ITERS_):
        for cp in copies(t % NB_, t):
            cp.wait()


def kernel(input):
    mesh = plsc.VectorSubcoreMesh(core_axis_name="c", subcore_axis_name="s")
    f = functools.partial(
        pl.kernel,
        out_type=jax.ShapeDtypeStruct((N_, NUM_CLASSES_), jnp.int32),
        mesh=mesh,
        compiler_params=pltpu.CompilerParams(use_tc_tiling_on_sc=False, needs_layout_passes=False),
        scratch_types=[
            pltpu.VMEM((RPW_,), jnp.int32),
            pltpu.VMEM((NB_, SUB_, NUM_CLASSES_), jnp.int32),
            pltpu.SemaphoreType.DMA,
            pltpu.SemaphoreType.DMA((NB_,)),
        ],
    )(_sc_onehot)
    zeros = jnp.zeros((NB_, SUB_, NUM_CLASSES_), jnp.int32)
    return f(input.astype(jnp.int32), zeros)
